# Initial kernel scaffold; baseline (speedup 1.0000x reference)
import functools

import jax
import jax.numpy as jnp
from jax import lax
from jax.experimental import pallas as pl
from jax.experimental.pallas import tpu as pltpu
from jax.experimental.pallas import tpu_sc as plsc

B = 16384
NF = 26
VOCAB = 100000
D = 32
L = 50
LP = 64
NO = NF + 1
NC = 2
NS = 16
NW = NC * NS
BPW = B // NW
C = 32
NCHUNK = BPW // C
LANES = 16

SID_N = C * NF
FIDX_N = C * NO
QID_N = C * LP


def _sc_body(sid_hbm, qid_hbm, stab_hbm, qtab_hbm, out_hbm,
             sid_v, qid_v, fidx_v, obuf, qrow, t0_v, sem):
    wid = lax.axis_index("s") * NC + lax.axis_index("c")
    base = wid * BPW

    pltpu.sync_copy(qtab_hbm.at[pl.ds(0, 1)], t0_v)
    t00 = t0_v[0, 0:16]
    t01 = t0_v[0, 16:32]
    iota = lax.iota(jnp.int32, LANES)

    def chunk_body(g, carry):
        b0 = base + g * C

        cp_sid = pltpu.async_copy(
            sid_hbm.at[pl.ds(b0 * NF, SID_N)], sid_v, sem)
        cp_qid = pltpu.async_copy(
            qid_hbm.at[pl.ds(b0 * LP, QID_N)], qid_v, sem)
        cp_sid.wait()
        cp_qid.wait()

        for s in range(FIDX_N // LANES):
            p = iota + (s * LANES)
            c_idx = p // NO
            i_idx = p - c_idx * NO
            src = jnp.minimum(p - c_idx, SID_N - 1)
            val = plsc.load_gather(sid_v, [src])
            f = val + i_idx * VOCAB
            f = jnp.where(i_idx < NF, f, jnp.zeros_like(f))
            fidx_v[pl.ds(s * LANES, LANES)] = f

        copies = []
        off = 0
        while off < FIDX_N:
            n = min(128, FIDX_N - off)
            copies.append(pltpu.async_copy(
                stab_hbm.at[fidx_v.at[pl.ds(off, n)]],
                obuf.at[pl.ds(off, n)], sem))
            off += n
        for off in range(0, QID_N, 128):
            copies.append(pltpu.async_copy(
                qtab_hbm.at[qid_v.at[pl.ds(off, 128)]],
                qrow.at[pl.ds(off, 128)], sem))
        for cp in copies:
            cp.wait()

        def acc_body(c, carry2):
            qb = c * LP
            acc0 = jnp.zeros((LANES,), jnp.float32)
            acc1 = jnp.zeros((LANES,), jnp.float32)
            for l in range(LP):
                acc0 = acc0 + qrow[qb + l, 0:16]
                acc1 = acc1 + qrow[qb + l, 16:32]
            npad = jnp.zeros((LANES,), jnp.int32)
            for j in range(LP // LANES):
                q = qid_v[pl.ds(qb + j * LANES, LANES)]
                npad = npad + plsc.all_reduce_population_count(q == 0)
            npf = npad.astype(jnp.float32)
            denom = (jnp.float32(LP) - npf) + jnp.float32(1e-16)
            orow = c * NO + NF
            obuf[orow, 0:16] = (acc0 - npf * t00) / denom
            obuf[orow, 16:32] = (acc1 - npf * t01) / denom
            return carry2

        lax.fori_loop(0, C, acc_body, 0)

        pltpu.async_copy(
            obuf, out_hbm.at[pl.ds(b0 * NO, FIDX_N)], sem).wait()
        return carry

    lax.fori_loop(0, NCHUNK, chunk_body, 0)


_sc_kernel = functools.partial(
    pl.kernel,
    out_type=jax.ShapeDtypeStruct((B * NO, D), jnp.float32),
    mesh=plsc.VectorSubcoreMesh(
        core_axis_name="c", subcore_axis_name="s",
        num_cores=NC, num_subcores=NS),
    scratch_types=[
        pltpu.VMEM((SID_N,), jnp.int32),
        pltpu.VMEM((QID_N,), jnp.int32),
        pltpu.VMEM((FIDX_N,), jnp.int32),
        pltpu.VMEM((FIDX_N, D), jnp.float32),
        pltpu.VMEM((QID_N, D), jnp.float32),
        pltpu.VMEM((1, D), jnp.float32),
        pltpu.SemaphoreType.DMA,
    ],
)(_sc_body)


@jax.jit
def kernel(sparse_ids, seq_ids, sparse_tables, seq_table):
    sid_flat = sparse_ids.reshape(B * NF)
    qid_flat = jnp.pad(seq_ids, ((0, 0), (0, LP - L))).reshape(B * LP)
    stab = sparse_tables.reshape(NF * VOCAB, D)
    out = _sc_kernel(sid_flat, qid_flat, stab, seq_table)
    return out.reshape(B, NO, D)


# speedup vs baseline: 1.3711x; 1.3711x over previous
import functools

import jax
import jax.numpy as jnp
from jax import lax
from jax.experimental import pallas as pl
from jax.experimental.pallas import tpu as pltpu
from jax.experimental.pallas import tpu_sc as plsc

B = 16384
NF = 26
VOCAB = 100000
D = 32
L = 50
LP = 64
NO = NF + 1
NC = 2
NS = 16
NW = NC * NS
BPW = B // NW
C = 32
NCHUNK = BPW // C
LANES = 16

SID_N = C * NF
FIDX_N = C * NO
QID_N = C * LP


def _sc_body(sid_hbm, qid_hbm, stab_hbm, qtab_hbm, out_hbm,
             sid_v, qid_v, fidx_v, obuf, qrow, t0_v, sem):
    wid = lax.axis_index("s") * NC + lax.axis_index("c")
    base = wid * BPW

    pltpu.sync_copy(qtab_hbm.at[pl.ds(0, 1)], t0_v)
    t00 = t0_v[0, 0:16]
    t01 = t0_v[0, 16:32]
    iota = lax.iota(jnp.int32, LANES)

    def chunk_body(g, carry):
        b0 = base + g * C

        cp_sid = pltpu.async_copy(
            sid_hbm.at[pl.ds(b0 * NF, SID_N)], sid_v, sem)
        cp_qid = pltpu.async_copy(
            qid_hbm.at[pl.ds(b0 * LP, QID_N)], qid_v, sem)
        cp_sid.wait()
        cp_qid.wait()

        for s in range(FIDX_N // LANES):
            p = iota + (s * LANES)
            c_idx = p // NO
            i_idx = p - c_idx * NO
            src = jnp.minimum(p - c_idx, SID_N - 1)
            val = plsc.load_gather(sid_v, [src])
            f = val + i_idx * VOCAB
            f = jnp.where(i_idx < NF, f, jnp.zeros_like(f))
            fidx_v[pl.ds(s * LANES, LANES)] = f

        copies = []
        off = 0
        while off < FIDX_N:
            n = min(128, FIDX_N - off)
            copies.append(pltpu.async_copy(
                stab_hbm.at[fidx_v.at[pl.ds(off, n)]],
                obuf.at[pl.ds(off, n)], sem))
            off += n
        for off in range(0, QID_N, 128):
            copies.append(pltpu.async_copy(
                qtab_hbm.at[qid_v.at[pl.ds(off, 128)]],
                qrow.at[pl.ds(off, 128)], sem))
        for cp in copies:
            cp.wait()

        def acc_body(c, carry2):
            qb = c * LP
            acc0 = jnp.zeros((LANES,), jnp.float32)
            acc1 = jnp.zeros((LANES,), jnp.float32)
            for l in range(LP):
                acc0 = acc0 + qrow[qb + l, 0:16]
                acc1 = acc1 + qrow[qb + l, 16:32]
            npad = jnp.zeros((LANES,), jnp.int32)
            for j in range(LP // LANES):
                q = qid_v[pl.ds(qb + j * LANES, LANES)]
                npad = npad + plsc.all_reduce_population_count(q == 0)
            npf = npad.astype(jnp.float32)
            denom = (jnp.float32(LP) - npf) + jnp.float32(1e-16)
            orow = c * NO + NF
            obuf[orow, 0:16] = (acc0 - npf * t00) / denom
            obuf[orow, 16:32] = (acc1 - npf * t01) / denom
            return carry2

        lax.fori_loop(0, C, acc_body, 0)

        pltpu.async_copy(
            obuf, out_hbm.at[pl.ds(b0 * NO, FIDX_N)], sem).wait()
        return carry

    lax.fori_loop(0, NCHUNK, chunk_body, 0)


_sc_kernel = functools.partial(
    pl.kernel,
    out_type=jax.ShapeDtypeStruct((B * NO, D), jnp.float32),
    mesh=plsc.VectorSubcoreMesh(
        core_axis_name="c", subcore_axis_name="s",
        num_cores=NC, num_subcores=NS),
    compiler_params=pltpu.CompilerParams(
        use_tc_tiling_on_sc=False, needs_layout_passes=False),
    scratch_types=[
        pltpu.VMEM((SID_N,), jnp.int32),
        pltpu.VMEM((QID_N,), jnp.int32),
        pltpu.VMEM((FIDX_N,), jnp.int32),
        pltpu.VMEM((FIDX_N, D), jnp.float32),
        pltpu.VMEM((QID_N, D), jnp.float32),
        pltpu.VMEM((1, D), jnp.float32),
        pltpu.SemaphoreType.DMA,
    ],
)(_sc_body)


@jax.jit
def kernel(sparse_ids, seq_ids, sparse_tables, seq_table):
    sid_flat = sparse_ids.reshape(B * NF)
    qid_flat = jnp.pad(seq_ids, ((0, 0), (0, LP - L))).reshape(B * LP)
    stab = sparse_tables.reshape(NF * VOCAB, D)
    out = _sc_kernel(sid_flat, qid_flat, stab, seq_table)
    return out.reshape(B, NO, D)


# speedup vs baseline: 2.0322x; 1.4821x over previous
import functools

import jax
import jax.numpy as jnp
from jax import lax
from jax.experimental import pallas as pl
from jax.experimental.pallas import tpu as pltpu
from jax.experimental.pallas import tpu_sc as plsc

B = 16384
NF = 26
VOCAB = 100000
D = 32
L = 50
LP = 56
NO = NF + 1
NC = 2
NS = 16
NW = NC * NS
BPW = B // NW
C = 32
NCHUNK = BPW // C
LANES = 16

SID_N = C * NF
FIDX_N = C * NO
QID_N = C * LP


def _sc_body(sid_hbm, qid_hbm, stab_hbm, qtab_hbm, out_hbm,
             sid_v, qid_v, fidx_v, obuf, qrow, t0_v, sem):
    wid = lax.axis_index("s") * NC + lax.axis_index("c")
    base = wid * BPW

    pltpu.sync_copy(qtab_hbm.at[pl.ds(0, 1)], t0_v)
    t00 = t0_v[0, 0:16]
    t01 = t0_v[0, 16:32]
    iota = lax.iota(jnp.int32, LANES)

    def chunk_body(g, carry):
        b0 = base + g * C

        cp_sid = pltpu.async_copy(
            sid_hbm.at[pl.ds(b0 * NF, SID_N)], sid_v, sem)
        cp_qid = pltpu.async_copy(
            qid_hbm.at[pl.ds(b0 * LP, QID_N)], qid_v, sem)
        cp_sid.wait()
        cp_qid.wait()

        for s in range(FIDX_N // LANES):
            p = iota + (s * LANES)
            c_idx = p // NO
            i_idx = p - c_idx * NO
            src = jnp.minimum(p - c_idx, SID_N - 1)
            val = plsc.load_gather(sid_v, [src])
            f = val + i_idx * VOCAB
            f = jnp.where(i_idx < NF, f, jnp.zeros_like(f))
            fidx_v[pl.ds(s * LANES, LANES)] = f

        copies = []
        off = 0
        while off < FIDX_N:
            n = min(128, FIDX_N - off)
            copies.append(pltpu.async_copy(
                stab_hbm.at[fidx_v.at[pl.ds(off, n)]],
                obuf.at[pl.ds(off, n)], sem))
            off += n
        for off in range(0, QID_N, 128):
            copies.append(pltpu.async_copy(
                qtab_hbm.at[qid_v.at[pl.ds(off, 128)]],
                qrow.at[pl.ds(off, 128)], sem))
        for cp in copies:
            cp.wait()

        def acc_body(c, carry2):
            qb = c * LP
            acc0 = jnp.zeros((LANES,), jnp.float32)
            acc1 = jnp.zeros((LANES,), jnp.float32)
            for l in range(LP):
                acc0 = acc0 + qrow[qb + l, 0:16]
                acc1 = acc1 + qrow[qb + l, 16:32]
            npad = jnp.zeros((LANES,), jnp.int32)
            for j in range(L // LANES):
                q = qid_v[pl.ds(qb + j * LANES, LANES)]
                npad = npad + plsc.all_reduce_population_count(q == 0)
            qt = qid_v[pl.ds(qb + LP - LANES, LANES)]
            npad = npad + plsc.all_reduce_population_count(
                (qt == 0) & (iota >= 8))
            npf = npad.astype(jnp.float32)
            denom = (jnp.float32(LP) - npf) + jnp.float32(1e-16)
            orow = c * NO + NF
            obuf[orow, 0:16] = (acc0 - npf * t00) / denom
            obuf[orow, 16:32] = (acc1 - npf * t01) / denom
            return carry2

        lax.fori_loop(0, C, acc_body, 0)

        pltpu.async_copy(
            obuf, out_hbm.at[pl.ds(b0 * NO, FIDX_N)], sem).wait()
        return carry

    lax.fori_loop(0, NCHUNK, chunk_body, 0)


_sc_kernel = functools.partial(
    pl.kernel,
    out_type=jax.ShapeDtypeStruct((B * NO, D), jnp.float32),
    mesh=plsc.VectorSubcoreMesh(
        core_axis_name="c", subcore_axis_name="s",
        num_cores=NC, num_subcores=NS),
    compiler_params=pltpu.CompilerParams(
        use_tc_tiling_on_sc=False, needs_layout_passes=False),
    scratch_types=[
        pltpu.VMEM((SID_N,), jnp.int32),
        pltpu.VMEM((QID_N,), jnp.int32),
        pltpu.VMEM((FIDX_N,), jnp.int32),
        pltpu.VMEM((FIDX_N, D), jnp.float32),
        pltpu.VMEM((QID_N, D), jnp.float32),
        pltpu.VMEM((1, D), jnp.float32),
        pltpu.SemaphoreType.DMA,
    ],
)(_sc_body)


@jax.jit
def kernel(sparse_ids, seq_ids, sparse_tables, seq_table):
    sid_flat = sparse_ids.reshape(B * NF)
    qid_flat = jnp.pad(seq_ids, ((0, 0), (0, LP - L))).reshape(B * LP)
    stab = sparse_tables.reshape(NF * VOCAB, D)
    out = _sc_kernel(sid_flat, qid_flat, stab, seq_table)
    return out.reshape(B, NO, D)
